# baseline (speedup 1.0000x reference)
import jax
import jax.numpy as jnp
from jax.experimental import pallas as pl

_TOP_K = 8
_NUM_EXPERTS = 64
_BLOCK_M = 1024
_QBITS = 25
_QSCALE = float(2 ** _QBITS)


def _router_kernel(x_ref, w_ref, ts_ref, idx_ref):
    x = x_ref[...]
    w = w_ref[...]
    scores = jax.lax.dot_general(
        x, w, (((1,), (1,)), ((), ())), preferred_element_type=jnp.float32
    )
    s = jax.nn.sigmoid(scores)
    lane = jax.lax.broadcasted_iota(jnp.int32, s.shape, 1)
    cur = s
    vals = []
    idxs = []
    for _ in range(_TOP_K):
        m = jnp.max(cur, axis=1, keepdims=True)
        eq = cur == m
        sel = jnp.min(jnp.where(eq, lane, _NUM_EXPERTS), axis=1, keepdims=True)
        vals.append(m)
        idxs.append(sel)
        cur = jnp.where(eq, -1.0, cur)
    v = jnp.concatenate(vals, axis=1)
    idx = jnp.concatenate(idxs, axis=1)
    denom = jnp.sum(v, axis=1, keepdims=True) + 1e-20
    ts_ref[...] = v / denom
    idx_ref[...] = idx


def kernel(x, gate_weight):
    n_tokens = x.shape[0]
    grid = (n_tokens // _BLOCK_M,)
    return pl.pallas_call(
        _router_kernel,
        grid=grid,
        in_specs=[
            pl.BlockSpec((_BLOCK_M, x.shape[1]), lambda i: (i, 0)),
            pl.BlockSpec(gate_weight.shape, lambda i: (0, 0)),
        ],
        out_specs=[
            pl.BlockSpec((_BLOCK_M, _TOP_K), lambda i: (i, 0)),
            pl.BlockSpec((_BLOCK_M, _TOP_K), lambda i: (i, 0)),
        ],
        out_shape=[
            jax.ShapeDtypeStruct((n_tokens, _TOP_K), jnp.float32),
            jax.ShapeDtypeStruct((n_tokens, _TOP_K), jnp.int32),
        ],
    )(x, gate_weight)


# speedup vs baseline: 1.1433x; 1.1433x over previous
import jax
import jax.numpy as jnp
from jax.experimental import pallas as pl

_TOP_K = 8
_NUM_EXPERTS = 64
_BLOCK_M = 1024
_QBITS = 25
_QSCALE = float(2 ** _QBITS)


def _router_kernel(x_ref, w_ref, ts_ref, idx_ref):
    x = x_ref[...]
    w = w_ref[...]
    scores = jax.lax.dot_general(
        x, w, (((1,), (1,)), ((), ())), preferred_element_type=jnp.float32
    )
    s = jax.nn.sigmoid(scores)
    lane = jax.lax.broadcasted_iota(jnp.int32, s.shape, 1)
    sbits = jax.lax.bitcast_convert_type(s, jnp.int32)
    kbits = ((sbits << 1) & ~(_NUM_EXPERTS - 1)) | ((_NUM_EXPERTS - 1) - lane)
    cur = jax.lax.bitcast_convert_type(kbits, jnp.float32)
    vals = []
    for _ in range(_TOP_K):
        m = jnp.max(cur, axis=1, keepdims=True)
        vals.append(m)
        cur = jnp.where(cur == m, -1.0, cur)
    k8 = jax.lax.bitcast_convert_type(jnp.concatenate(vals, axis=1), jnp.int32)
    idx = (_NUM_EXPERTS - 1) - (k8 & (_NUM_EXPERTS - 1))
    v = jax.lax.bitcast_convert_type(
        (k8 & ~(_NUM_EXPERTS - 1)) >> 1, jnp.float32
    )
    denom = jnp.sum(v, axis=1, keepdims=True) + 1e-20
    ts_ref[...] = v / denom
    idx_ref[...] = idx


def kernel(x, gate_weight):
    n_tokens = x.shape[0]
    grid = (n_tokens // _BLOCK_M,)
    return pl.pallas_call(
        _router_kernel,
        grid=grid,
        in_specs=[
            pl.BlockSpec((_BLOCK_M, x.shape[1]), lambda i: (i, 0)),
            pl.BlockSpec(gate_weight.shape, lambda i: (0, 0)),
        ],
        out_specs=[
            pl.BlockSpec((_BLOCK_M, _TOP_K), lambda i: (i, 0)),
            pl.BlockSpec((_BLOCK_M, _TOP_K), lambda i: (i, 0)),
        ],
        out_shape=[
            jax.ShapeDtypeStruct((n_tokens, _TOP_K), jnp.float32),
            jax.ShapeDtypeStruct((n_tokens, _TOP_K), jnp.int32),
        ],
    )(x, gate_weight)
